# baseline (speedup 1.0000x reference)
import functools

import jax
import jax.numpy as jnp
from jax import lax
from jax.experimental import pallas as pl
from jax.experimental.pallas import tpu as pltpu
from jax.experimental.pallas import tpu_sc as plsc

NC = 2
NS = 16
NW = NC * NS
CHUNK = 128


def _node_mlp_body(x_ref, w1_ref, b1_ref, w2_ref, b2_ref, o_ref):
    h = jnp.maximum(jnp.dot(x_ref[...], w1_ref[...],
                            preferred_element_type=jnp.float32) + b1_ref[...], 0.0)
    o_ref[...] = jnp.maximum(jnp.dot(h, w2_ref[...],
                                     preferred_element_type=jnp.float32)
                             + b2_ref[...], 0.0)


def _node_mlp(x, w1, b1, w2, b2, block=1000):
    n, d = x.shape
    h1 = w1.shape[1]
    h2 = w2.shape[1]
    grid = n // block
    return pl.pallas_call(
        _node_mlp_body,
        grid=(grid,),
        in_specs=[
            pl.BlockSpec((block, d), lambda i: (i, 0)),
            pl.BlockSpec((d, h1), lambda i: (0, 0)),
            pl.BlockSpec((1, h1), lambda i: (0, 0)),
            pl.BlockSpec((h1, h2), lambda i: (0, 0)),
            pl.BlockSpec((1, h2), lambda i: (0, 0)),
        ],
        out_specs=pl.BlockSpec((block, h2), lambda i: (i, 0)),
        out_shape=jax.ShapeDtypeStruct((n, h2), jnp.float32),
    )(x, w1, b1.reshape(1, h1), w2, b2.reshape(1, h2))


def _edge_mlp_body(gp_ref, gs_ref, w1_ref, kp_ref, kc_ref, pw2_ref, pb2_ref,
                   cw2_ref, cb2_ref, sp_ref, sc_ref):
    bf = jnp.bfloat16
    h1 = pw2_ref.shape[0]
    x = jnp.concatenate([gp_ref[...], gs_ref[...]], axis=1).astype(bf)
    h = jnp.dot(x, w1_ref[...], preferred_element_type=jnp.float32)
    hp = jnp.maximum(h[:, :h1] + kp_ref[...], 0.0).astype(bf)
    hc = jnp.maximum(h[:, h1:] + kc_ref[...], 0.0).astype(bf)
    sp_ref[...] = jnp.maximum(
        jnp.dot(hp, pw2_ref[...].astype(bf), preferred_element_type=jnp.float32)
        + pb2_ref[...], 0.0)
    sc_ref[...] = jnp.maximum(
        jnp.dot(hc, cw2_ref[...].astype(bf), preferred_element_type=jnp.float32)
        + cb2_ref[...], 0.0)


def _edge_mlp(gp, gs, w1cat, kp, kc, pw2, pb2, cw2, cb2, ne, blk0=0,
              block=10000):
    e, d = gp.shape
    h1 = pw2.shape[0]
    h2 = pw2.shape[1]
    grid = ne // block
    wspec = lambda shape: pl.BlockSpec(shape, lambda i: (0, 0))
    return pl.pallas_call(
        _edge_mlp_body,
        grid=(grid,),
        in_specs=[
            pl.BlockSpec((block, d), lambda i: (blk0 + i, 0)),
            pl.BlockSpec((block, d), lambda i: (blk0 + i, 0)),
            wspec((2 * d, 2 * h1)), wspec((1, h1)), wspec((1, h1)),
            wspec((h1, h2)), wspec((1, h2)),
            wspec((h1, h2)), wspec((1, h2)),
        ],
        out_specs=[
            pl.BlockSpec((block, h2), lambda i: (i, 0)),
            pl.BlockSpec((block, h2), lambda i: (i, 0)),
        ],
        out_shape=[
            jax.ShapeDtypeStruct((ne, h2), jnp.float32),
            jax.ShapeDtypeStruct((ne, h2), jnp.float32),
        ],
    )(gp, gs, w1cat, kp, kc, pw2, pb2, cw2, cb2)


def _aggr_body(h_ref, sump_ref, sumc_ref, cnts_ref, cntp_ref, rm_ref, lm_ref,
               st_ref, et_ref, a1_ref, b1_ref, a2_ref, b2_ref, o_ref):
    h = h_ref[...]
    sp = sump_ref[...] / jnp.maximum(cnts_ref[...][:, 0:1], 1.0) \
        + rm_ref[...] * st_ref[...]
    sc = sumc_ref[...] / jnp.maximum(cntp_ref[...][:, 0:1], 1.0) \
        + lm_ref[...] * et_ref[...]
    x = jnp.concatenate([h, sp, sc], axis=1)
    pre = jnp.maximum(jnp.dot(x, a1_ref[...],
                              preferred_element_type=jnp.float32) + b1_ref[...], 0.0)
    out = jnp.maximum(jnp.dot(pre, a2_ref[...],
                              preferred_element_type=jnp.float32) + b2_ref[...], 0.0)
    o_ref[...] = h + out


def _aggr_mlp(h, sump, sumc, cnts, cntp, rm, lm, st, et,
              a1, b1, a2, b2, block=1000):
    n, d = h.shape
    h1 = a1.shape[1]
    h2 = a2.shape[1]
    grid = n // block
    wspec = lambda shape: pl.BlockSpec(shape, lambda i: (0, 0))
    bspec = pl.BlockSpec((block, d), lambda i: (i, 0))
    return pl.pallas_call(
        _aggr_body,
        grid=(grid,),
        in_specs=[
            bspec, bspec, bspec,
            pl.BlockSpec((block, 128), lambda i: (i, 0)),
            pl.BlockSpec((block, 128), lambda i: (i, 0)),
            pl.BlockSpec((block, 1), lambda i: (i, 0)),
            pl.BlockSpec((block, 1), lambda i: (i, 0)),
            wspec((1, d)), wspec((1, d)),
            wspec((3 * d, h1)), wspec((1, h1)),
            wspec((h1, h2)), wspec((1, h2)),
        ],
        out_specs=pl.BlockSpec((block, h2), lambda i: (i, 0)),
        out_shape=jax.ShapeDtypeStruct((n, h2), jnp.float32),
    )(h, sump, sumc, cnts, cntp, rm, lm, st, et, a1, b1, a2, b2)


def _sc_mesh():
    return plsc.VectorSubcoreMesh(core_axis_name="c", subcore_axis_name="s",
                                  num_cores=NC, num_subcores=NS)


def _gather_pallas(hidden, sidx, pidx, lo, hi):
    n, d = hidden.shape
    dt = hidden.dtype
    e = hi - lo
    per_w = e // NW
    nchunks = per_w // CHUNK
    nmain = nchunks - (nchunks % 2)
    rem = per_w - nchunks * CHUNK

    @functools.partial(
        pl.kernel,
        out_type=[jax.ShapeDtypeStruct((e, d), dt),
                  jax.ShapeDtypeStruct((e, d), dt)],
        mesh=_sc_mesh(),
        scratch_types=[
            pltpu.VMEM((CHUNK,), jnp.int32),
            pltpu.VMEM((CHUNK,), jnp.int32),
            pltpu.VMEM((CHUNK,), jnp.int32),
            pltpu.VMEM((CHUNK,), jnp.int32),
            pltpu.VMEM((CHUNK, d), dt),
            pltpu.VMEM((CHUNK, d), dt),
            pltpu.VMEM((CHUNK, d), dt),
            pltpu.VMEM((CHUNK, d), dt),
            pltpu.VMEM((max(rem, 8),), jnp.int32),
            pltpu.VMEM((max(rem, 8),), jnp.int32),
            pltpu.VMEM((max(rem, 8), d), dt),
            pltpu.VMEM((max(rem, 8), d), dt),
            pltpu.SemaphoreType.DMA,
            pltpu.SemaphoreType.DMA,
            pltpu.SemaphoreType.DMA,
        ],
    )
    def k(hid_ref, sidx_ref, pidx_ref, gs_ref, gp_ref,
          is0, is1, ip0, ip1, rs0, rs1, rp0, rp1,
          ris, rip, rrs, rrp, semi, semg, semw):
        wid = lax.axis_index("s") * NC + lax.axis_index("c")
        base = wid * per_w
        i_s = [is0, is1]
        i_p = [ip0, ip1]
        r_s = [rs0, rs1]
        r_p = [rp0, rp1]

        def idx_start(j, b):
            off = lo + base + j * CHUNK
            pltpu.async_copy(sidx_ref.at[pl.ds(off, CHUNK)], i_s[b], semi)
            pltpu.async_copy(pidx_ref.at[pl.ds(off, CHUNK)], i_p[b], semi)

        def idx_wait():
            pltpu.make_async_copy(sidx_ref.at[pl.ds(0, CHUNK)], i_s[0], semi).wait()
            pltpu.make_async_copy(pidx_ref.at[pl.ds(0, CHUNK)], i_p[0], semi).wait()

        def gather_start(b):
            pltpu.async_copy(hid_ref.at[i_s[b]], r_s[b], semg)
            pltpu.async_copy(hid_ref.at[i_p[b]], r_p[b], semg)

        def gather_wait(b):
            pltpu.make_async_copy(hid_ref.at[i_s[b]], r_s[b], semg).wait()
            pltpu.make_async_copy(hid_ref.at[i_p[b]], r_p[b], semg).wait()

        def write_start(j, b):
            off = base + j * CHUNK
            pltpu.async_copy(r_s[b], gs_ref.at[pl.ds(off, CHUNK)], semw)
            pltpu.async_copy(r_p[b], gp_ref.at[pl.ds(off, CHUNK)], semw)

        def write_wait():
            pltpu.make_async_copy(r_s[0], gs_ref.at[pl.ds(0, CHUNK)], semw).wait()
            pltpu.make_async_copy(r_p[0], gp_ref.at[pl.ds(0, CHUNK)], semw).wait()

        idx_start(0, 0)

        def body(t, carry):
            j0 = 2 * t
            idx_wait()

            @pl.when(t > 0)
            def _():
                write_wait()

            gather_start(0)
            idx_start(j0 + 1, 1)
            gather_wait(0)
            write_start(j0, 0)

            idx_wait()

            @pl.when(t > 0)
            def _():
                write_wait()

            gather_start(1)

            @pl.when(t < nmain // 2 - 1)
            def _():
                idx_start(j0 + 2, 0)

            gather_wait(1)
            write_start(j0 + 1, 1)
            return carry

        lax.fori_loop(0, nmain // 2, body, 0)
        for j in range(nmain, nchunks):
            idx_start(j, 0)
            idx_wait()
            write_wait()
            gather_start(0)
            gather_wait(0)
            write_start(j, 0)
        write_wait()
        write_wait()
        if rem:
            offr = base + nchunks * CHUNK
            pltpu.sync_copy(sidx_ref.at[pl.ds(lo + offr, rem)], ris.at[pl.ds(0, rem)])
            pltpu.sync_copy(pidx_ref.at[pl.ds(lo + offr, rem)], rip.at[pl.ds(0, rem)])
            pltpu.async_copy(hid_ref.at[ris.at[pl.ds(0, rem)]],
                             rrs.at[pl.ds(0, rem)], semg).wait()
            pltpu.async_copy(hid_ref.at[rip.at[pl.ds(0, rem)]],
                             rrp.at[pl.ds(0, rem)], semg).wait()
            pltpu.sync_copy(rrs.at[pl.ds(0, rem)], gs_ref.at[pl.ds(offr, rem)])
            pltpu.sync_copy(rrp.at[pl.ds(0, rem)], gp_ref.at[pl.ds(offr, rem)])

    return k(hidden, sidx, pidx)


def _scatter_pallas(sp_edge, sc_edge, sidx, pidx, initp, initc, lo):
    e, d = sp_edge.shape
    n = initp.shape[0]
    per_t = e // NS
    nchunks = per_t // CHUNK
    rem = per_t - nchunks * CHUNK
    nzt = 10
    rpt = n // nzt

    @functools.partial(
        pl.kernel,
        out_type=[jax.ShapeDtypeStruct((n, d), jnp.float32),
                  jax.ShapeDtypeStruct((n, d), jnp.float32)],
        mesh=_sc_mesh(),
        scratch_types=[
            pltpu.VMEM_SHARED((n, d), jnp.float32),
            pltpu.VMEM((CHUNK,), jnp.int32),
            pltpu.VMEM((CHUNK,), jnp.int32),
            pltpu.VMEM((CHUNK, d), jnp.float32),
            pltpu.VMEM((CHUNK, d), jnp.float32),
            pltpu.VMEM((max(rem, 8),), jnp.int32),
            pltpu.VMEM((max(rem, 8), d), jnp.float32),
            pltpu.SemaphoreType.DMA,
            pltpu.SemaphoreType.DMA,
        ],
    )
    def k(sp_ref, sc_ref, sidx_ref, pidx_ref, zp_ref, zc_ref, outp_ref, outc_ref,
          acc, idx0, idx1, val0, val1, ridx, rval, semi, sema):
        c = lax.axis_index("c")
        s = lax.axis_index("s")
        row0 = s * rpt

        @pl.when((s < nzt) & (c == 0))
        def _():
            pltpu.sync_copy(zp_ref.at[pl.ds(row0, rpt)], acc.at[pl.ds(row0, rpt)])

        @pl.when((s < nzt) & (c == 1))
        def _():
            pltpu.sync_copy(zc_ref.at[pl.ds(row0, rpt)], acc.at[pl.ds(row0, rpt)])

        plsc.subcore_barrier()
        base = s * per_t

        def run(val_ref, i_ref):
            idx_b = [idx0, idx1]
            val_b = [val0, val1]

            def copy_start(j, b):
                off = base + j * CHUNK
                pltpu.async_copy(i_ref.at[pl.ds(lo + off, CHUNK)], idx_b[b], semi)
                pltpu.async_copy(val_ref.at[pl.ds(off, CHUNK)], val_b[b], semi)

            def copy_wait():
                pltpu.make_async_copy(i_ref.at[pl.ds(0, CHUNK)], idx_b[0], semi).wait()
                pltpu.make_async_copy(val_ref.at[pl.ds(0, CHUNK)], val_b[0], semi).wait()

            def add_start(b):
                pltpu.async_copy(val_b[b], acc.at[idx_b[b]], sema, add=True)

            def add_wait(b):
                pltpu.make_async_copy(val_b[b], acc.at[idx_b[b]], sema).wait()

            copy_start(0, 0)

            def body(t, carry):
                j0 = 2 * t
                copy_wait()
                add_start(0)

                @pl.when(t > 0)
                def _():
                    add_wait(1)

                copy_start(j0 + 1, 1)
                copy_wait()
                add_start(1)

                @pl.when(t < nchunks // 2 - 1)
                def _():
                    add_wait(0)
                    copy_start(j0 + 2, 0)

                return carry

            lax.fori_loop(0, nchunks // 2, body, 0)
            add_wait(0)
            add_wait(1)
            if rem:
                offr = base + nchunks * CHUNK
                pltpu.sync_copy(i_ref.at[pl.ds(lo + offr, rem)], ridx)
                pltpu.sync_copy(val_ref.at[pl.ds(offr, rem)], rval)
                pltpu.sync_copy(rval, acc.at[ridx], add=True)

        @pl.when(c == 0)
        def _():
            run(sp_ref, sidx_ref)

        @pl.when(c == 1)
        def _():
            run(sc_ref, pidx_ref)

        plsc.subcore_barrier()

        @pl.when((s < nzt) & (c == 0))
        def _():
            pltpu.sync_copy(acc.at[pl.ds(row0, rpt)], outp_ref.at[pl.ds(row0, rpt)])

        @pl.when((s < nzt) & (c == 1))
        def _():
            pltpu.sync_copy(acc.at[pl.ds(row0, rpt)], outc_ref.at[pl.ds(row0, rpt)])

    return k(sp_edge, sc_edge, sidx, pidx, initp, initc)


def _counts_pallas(sidx, pidx, zeros_n8, ones_c8):
    e = sidx.shape[0]
    n, d8 = zeros_n8.shape
    per_t = e // NS
    nchunks = per_t // CHUNK
    rem = per_t - nchunks * CHUNK
    nzt = 10
    rpt = n // nzt

    @functools.partial(
        pl.kernel,
        out_type=[jax.ShapeDtypeStruct((n, d8), jnp.float32),
                  jax.ShapeDtypeStruct((n, d8), jnp.float32)],
        mesh=_sc_mesh(),
        scratch_types=[
            pltpu.VMEM_SHARED((n, d8), jnp.float32),
            pltpu.VMEM((CHUNK,), jnp.int32),
            pltpu.VMEM((CHUNK,), jnp.int32),
            pltpu.VMEM((CHUNK, d8), jnp.float32),
            pltpu.VMEM((max(rem, 8),), jnp.int32),
            pltpu.SemaphoreType.DMA,
            pltpu.SemaphoreType.DMA,
        ],
    )
    def k(sidx_ref, pidx_ref, z_ref, ones_ref, outs_ref, outp_ref,
          acc, idx0, idx1, ones_v, ridx, semi, sema):
        c = lax.axis_index("c")
        s = lax.axis_index("s")
        row0 = s * rpt

        @pl.when(s < nzt)
        def _():
            pltpu.sync_copy(z_ref.at[pl.ds(row0, rpt)], acc.at[pl.ds(row0, rpt)])

        pltpu.sync_copy(ones_ref, ones_v)
        plsc.subcore_barrier()
        base = s * per_t

        def run(i_ref):
            idx_b = [idx0, idx1]

            def copy_start(j, b):
                off = base + j * CHUNK
                pltpu.async_copy(i_ref.at[pl.ds(off, CHUNK)], idx_b[b], semi)

            def copy_wait():
                pltpu.make_async_copy(i_ref.at[pl.ds(0, CHUNK)], idx_b[0], semi).wait()

            def add_start(b):
                pltpu.async_copy(ones_v, acc.at[idx_b[b]], sema, add=True)

            def add_wait(b):
                pltpu.make_async_copy(ones_v, acc.at[idx_b[b]], sema).wait()

            copy_start(0, 0)

            def body(t, carry):
                j0 = 2 * t
                copy_wait()
                add_start(0)

                @pl.when(t > 0)
                def _():
                    add_wait(1)

                copy_start(j0 + 1, 1)
                copy_wait()
                add_start(1)

                @pl.when(t < nchunks // 2 - 1)
                def _():
                    add_wait(0)
                    copy_start(j0 + 2, 0)

                return carry

            lax.fori_loop(0, nchunks // 2, body, 0)
            add_wait(0)
            add_wait(1)
            if rem:
                offr = base + nchunks * CHUNK
                pltpu.sync_copy(i_ref.at[pl.ds(offr, rem)], ridx)
                pltpu.sync_copy(ones_v.at[pl.ds(0, rem)], acc.at[ridx], add=True)

        @pl.when(c == 0)
        def _():
            run(sidx_ref)

        @pl.when(c == 1)
        def _():
            run(pidx_ref)

        plsc.subcore_barrier()

        @pl.when((s < nzt) & (c == 0))
        def _():
            pltpu.sync_copy(acc.at[pl.ds(row0, rpt)], outs_ref.at[pl.ds(row0, rpt)])

        @pl.when((s < nzt) & (c == 1))
        def _():
            pltpu.sync_copy(acc.at[pl.ds(row0, rpt)], outp_ref.at[pl.ds(row0, rpt)])

    return k(sidx, pidx, zeros_n8, ones_c8)


def kernel(batch_token, self_idx_batch, parent_idx_batch, root_mask, leaf_mask,
           start_token, end_token,
           V_W1, V_b1, V_W2, V_b2,
           E_W1, E_b1, E_W2, E_b2,
           p_W1, p_b1, p_W2, p_b2,
           c_W1, c_b1, c_W2, c_b2,
           aggr_W1, aggr_b1, aggr_W2, aggr_b2):
    n, d = batch_token.shape
    e = self_idx_batch.shape[0]
    num_hops = 2

    edge_in = jnp.maximum(jnp.maximum(jnp.ones((1, 1), jnp.float32) @ E_W1 + E_b1, 0.0)
                          @ E_W2 + E_b2, 0.0)
    edge_out = jnp.maximum(jnp.maximum(jnp.zeros((1, 1), jnp.float32) @ E_W1 + E_b1, 0.0)
                           @ E_W2 + E_b2, 0.0)
    kp = (edge_out @ p_W1[2 * d:] + p_b1).reshape(1, -1)
    kc = (edge_in @ c_W1[2 * d:] + c_b1).reshape(1, -1)

    w1cat = jnp.concatenate(
        [jnp.concatenate([p_W1[:d], p_W1[d:2 * d]], axis=0),
         jnp.concatenate([c_W1[d:2 * d], c_W1[:d]], axis=0)],
        axis=1).astype(jnp.bfloat16)

    hidden = _node_mlp(batch_token, V_W1, V_b1, V_W2, V_b2)

    zeros_nd = jnp.zeros((n, d), jnp.float32)
    ones_c = jnp.ones((CHUNK, d), jnp.float32)
    cnt_s, cnt_p = _counts_pallas(self_idx_batch, parent_idx_batch,
                                  zeros_nd, ones_c)

    rm = root_mask.reshape(n, 1)
    lm = leaf_mask.reshape(n, 1)
    st = start_token.reshape(1, d)
    et = end_token.reshape(1, d)
    pb2 = p_b2.reshape(1, -1)
    cb2 = c_b2.reshape(1, -1)

    e2 = e // 2
    for _ in range(num_hops):
        gs, gp = _gather_pallas(hidden, self_idx_batch, parent_idx_batch, 0, e)
        sp_a, sc_a = _edge_mlp(gp, gs, w1cat, kp, kc, p_W2, pb2, c_W2, cb2,
                               ne=e2, blk0=0)
        sp_b, sc_b = _edge_mlp(gp, gs, w1cat, kp, kc, p_W2, pb2, c_W2, cb2,
                               ne=e2, blk0=e2 // 10000)
        sump_a, sumc_a = _scatter_pallas(sp_a, sc_a, self_idx_batch,
                                         parent_idx_batch, zeros_nd, zeros_nd, 0)
        sum_p, sum_c = _scatter_pallas(sp_b, sc_b, self_idx_batch,
                                       parent_idx_batch, sump_a, sumc_a, e2)
        hidden = _aggr_mlp(hidden, sum_p, sum_c, cnt_s, cnt_p, rm, lm, st, et,
                           aggr_W1, aggr_b1.reshape(1, -1), aggr_W2,
                           aggr_b2.reshape(1, -1))
    return hidden


# speedup vs baseline: 1.0597x; 1.0597x over previous
import functools

import jax
import jax.numpy as jnp
from jax import lax
from jax.experimental import pallas as pl
from jax.experimental.pallas import tpu as pltpu
from jax.experimental.pallas import tpu_sc as plsc

NC = 2
NS = 16
NW = NC * NS
CHUNK = 128


def _node_mlp_body(x_ref, w1_ref, b1_ref, w2_ref, b2_ref, o_ref):
    h = jnp.maximum(jnp.dot(x_ref[...], w1_ref[...],
                            preferred_element_type=jnp.float32) + b1_ref[...], 0.0)
    o_ref[...] = jnp.maximum(jnp.dot(h, w2_ref[...],
                                     preferred_element_type=jnp.float32)
                             + b2_ref[...], 0.0)


def _node_mlp(x, w1, b1, w2, b2, block=1000):
    n, d = x.shape
    h1 = w1.shape[1]
    h2 = w2.shape[1]
    grid = n // block
    return pl.pallas_call(
        _node_mlp_body,
        grid=(grid,),
        in_specs=[
            pl.BlockSpec((block, d), lambda i: (i, 0)),
            pl.BlockSpec((d, h1), lambda i: (0, 0)),
            pl.BlockSpec((1, h1), lambda i: (0, 0)),
            pl.BlockSpec((h1, h2), lambda i: (0, 0)),
            pl.BlockSpec((1, h2), lambda i: (0, 0)),
        ],
        out_specs=pl.BlockSpec((block, h2), lambda i: (i, 0)),
        out_shape=jax.ShapeDtypeStruct((n, h2), jnp.float32),
    )(x, w1, b1.reshape(1, h1), w2, b2.reshape(1, h2))


def _edge_mlp_body(gp_ref, gs_ref, w1_ref, kp_ref, kc_ref, pw2_ref, pb2_ref,
                   cw2_ref, cb2_ref, sp_ref, sc_ref):
    bf = jnp.bfloat16
    h1 = pw2_ref.shape[0]
    x = jnp.concatenate([gp_ref[...], gs_ref[...]], axis=1).astype(bf)
    h = jnp.dot(x, w1_ref[...], preferred_element_type=jnp.float32)
    hp = jnp.maximum(h[:, :h1] + kp_ref[...], 0.0).astype(bf)
    hc = jnp.maximum(h[:, h1:] + kc_ref[...], 0.0).astype(bf)
    sp_ref[...] = jnp.maximum(
        jnp.dot(hp, pw2_ref[...].astype(bf), preferred_element_type=jnp.float32)
        + pb2_ref[...], 0.0)
    sc_ref[...] = jnp.maximum(
        jnp.dot(hc, cw2_ref[...].astype(bf), preferred_element_type=jnp.float32)
        + cb2_ref[...], 0.0)


def _edge_mlp(gp, gs, w1cat, kp, kc, pw2, pb2, cw2, cb2, block=10000):
    e, d = gp.shape
    h1 = pw2.shape[0]
    h2 = pw2.shape[1]
    grid = e // block
    wspec = lambda shape: pl.BlockSpec(shape, lambda i: (0, 0))
    return pl.pallas_call(
        _edge_mlp_body,
        grid=(grid,),
        in_specs=[
            pl.BlockSpec((block, d), lambda i: (i, 0)),
            pl.BlockSpec((block, d), lambda i: (i, 0)),
            wspec((2 * d, 2 * h1)), wspec((1, h1)), wspec((1, h1)),
            wspec((h1, h2)), wspec((1, h2)),
            wspec((h1, h2)), wspec((1, h2)),
        ],
        out_specs=[
            pl.BlockSpec((block, h2), lambda i: (i, 0)),
            pl.BlockSpec((block, h2), lambda i: (i, 0)),
        ],
        out_shape=[
            jax.ShapeDtypeStruct((e, h2), jnp.float32),
            jax.ShapeDtypeStruct((e, h2), jnp.float32),
        ],
    )(gp, gs, w1cat, kp, kc, pw2, pb2, cw2, cb2)


def _aggr_body(h_ref, sump_ref, sumc_ref, cnts_ref, cntp_ref, rm_ref, lm_ref,
               st_ref, et_ref, a1_ref, b1_ref, a2_ref, b2_ref, o_ref):
    h = h_ref[...]
    sp = sump_ref[...] / jnp.maximum(cnts_ref[...][:, 0:1], 1.0) \
        + rm_ref[...] * st_ref[...]
    sc = sumc_ref[...] / jnp.maximum(cntp_ref[...][:, 0:1], 1.0) \
        + lm_ref[...] * et_ref[...]
    x = jnp.concatenate([h, sp, sc], axis=1)
    pre = jnp.maximum(jnp.dot(x, a1_ref[...],
                              preferred_element_type=jnp.float32) + b1_ref[...], 0.0)
    out = jnp.maximum(jnp.dot(pre, a2_ref[...],
                              preferred_element_type=jnp.float32) + b2_ref[...], 0.0)
    o_ref[...] = h + out


def _aggr_mlp(h, sump, sumc, cnts, cntp, rm, lm, st, et,
              a1, b1, a2, b2, block=1000):
    n, d = h.shape
    h1 = a1.shape[1]
    h2 = a2.shape[1]
    grid = n // block
    wspec = lambda shape: pl.BlockSpec(shape, lambda i: (0, 0))
    bspec = pl.BlockSpec((block, d), lambda i: (i, 0))
    return pl.pallas_call(
        _aggr_body,
        grid=(grid,),
        in_specs=[
            bspec, bspec, bspec,
            pl.BlockSpec((block, 128), lambda i: (i, 0)),
            pl.BlockSpec((block, 128), lambda i: (i, 0)),
            pl.BlockSpec((block, 1), lambda i: (i, 0)),
            pl.BlockSpec((block, 1), lambda i: (i, 0)),
            wspec((1, d)), wspec((1, d)),
            wspec((3 * d, h1)), wspec((1, h1)),
            wspec((h1, h2)), wspec((1, h2)),
        ],
        out_specs=pl.BlockSpec((block, h2), lambda i: (i, 0)),
        out_shape=jax.ShapeDtypeStruct((n, h2), jnp.float32),
    )(h, sump, sumc, cnts, cntp, rm, lm, st, et, a1, b1, a2, b2)


def _sc_mesh():
    return plsc.VectorSubcoreMesh(core_axis_name="c", subcore_axis_name="s",
                                  num_cores=NC, num_subcores=NS)


def _gather_pallas(hidden, sidx, pidx):
    n, d = hidden.shape
    dt = hidden.dtype
    e = sidx.shape[0]
    per_w = e // NW
    nchunks = per_w // CHUNK
    nmain = nchunks - (nchunks % 2)
    rem = per_w - nchunks * CHUNK

    @functools.partial(
        pl.kernel,
        out_type=[jax.ShapeDtypeStruct((e, d), dt),
                  jax.ShapeDtypeStruct((e, d), dt)],
        mesh=_sc_mesh(),
        scratch_types=[
            pltpu.VMEM((CHUNK,), jnp.int32),
            pltpu.VMEM((CHUNK,), jnp.int32),
            pltpu.VMEM((CHUNK,), jnp.int32),
            pltpu.VMEM((CHUNK,), jnp.int32),
            pltpu.VMEM((CHUNK, d), dt),
            pltpu.VMEM((CHUNK, d), dt),
            pltpu.VMEM((CHUNK, d), dt),
            pltpu.VMEM((CHUNK, d), dt),
            pltpu.VMEM((max(rem, 8),), jnp.int32),
            pltpu.VMEM((max(rem, 8),), jnp.int32),
            pltpu.VMEM((max(rem, 8), d), dt),
            pltpu.VMEM((max(rem, 8), d), dt),
            pltpu.SemaphoreType.DMA,
            pltpu.SemaphoreType.DMA,
            pltpu.SemaphoreType.DMA,
        ],
    )
    def k(hid_ref, sidx_ref, pidx_ref, gs_ref, gp_ref,
          is0, is1, ip0, ip1, rs0, rs1, rp0, rp1,
          ris, rip, rrs, rrp, semi, semg, semw):
        wid = lax.axis_index("s") * NC + lax.axis_index("c")
        base = wid * per_w
        i_s = [is0, is1]
        i_p = [ip0, ip1]
        r_s = [rs0, rs1]
        r_p = [rp0, rp1]

        def idx_start(j, b):
            off = base + j * CHUNK
            pltpu.async_copy(sidx_ref.at[pl.ds(off, CHUNK)], i_s[b], semi)
            pltpu.async_copy(pidx_ref.at[pl.ds(off, CHUNK)], i_p[b], semi)

        def idx_wait():
            pltpu.make_async_copy(sidx_ref.at[pl.ds(0, CHUNK)], i_s[0], semi).wait()
            pltpu.make_async_copy(pidx_ref.at[pl.ds(0, CHUNK)], i_p[0], semi).wait()

        def gather_start(b):
            pltpu.async_copy(hid_ref.at[i_s[b]], r_s[b], semg)
            pltpu.async_copy(hid_ref.at[i_p[b]], r_p[b], semg)

        def gather_wait(b):
            pltpu.make_async_copy(hid_ref.at[i_s[b]], r_s[b], semg).wait()
            pltpu.make_async_copy(hid_ref.at[i_p[b]], r_p[b], semg).wait()

        def write_start(j, b):
            off = base + j * CHUNK
            pltpu.async_copy(r_s[b], gs_ref.at[pl.ds(off, CHUNK)], semw)
            pltpu.async_copy(r_p[b], gp_ref.at[pl.ds(off, CHUNK)], semw)

        def write_wait():
            pltpu.make_async_copy(r_s[0], gs_ref.at[pl.ds(0, CHUNK)], semw).wait()
            pltpu.make_async_copy(r_p[0], gp_ref.at[pl.ds(0, CHUNK)], semw).wait()

        idx_start(0, 0)

        def body(t, carry):
            j0 = 2 * t
            idx_wait()

            @pl.when(t > 0)
            def _():
                write_wait()

            gather_start(0)
            idx_start(j0 + 1, 1)
            gather_wait(0)
            write_start(j0, 0)

            idx_wait()

            @pl.when(t > 0)
            def _():
                write_wait()

            gather_start(1)

            @pl.when(t < nmain // 2 - 1)
            def _():
                idx_start(j0 + 2, 0)

            gather_wait(1)
            write_start(j0 + 1, 1)
            return carry

        lax.fori_loop(0, nmain // 2, body, 0)
        for j in range(nmain, nchunks):
            idx_start(j, 0)
            idx_wait()
            write_wait()
            gather_start(0)
            gather_wait(0)
            write_start(j, 0)
        write_wait()
        write_wait()
        if rem:
            offr = base + nchunks * CHUNK
            pltpu.sync_copy(sidx_ref.at[pl.ds(offr, rem)], ris.at[pl.ds(0, rem)])
            pltpu.sync_copy(pidx_ref.at[pl.ds(offr, rem)], rip.at[pl.ds(0, rem)])
            pltpu.async_copy(hid_ref.at[ris.at[pl.ds(0, rem)]],
                             rrs.at[pl.ds(0, rem)], semg).wait()
            pltpu.async_copy(hid_ref.at[rip.at[pl.ds(0, rem)]],
                             rrp.at[pl.ds(0, rem)], semg).wait()
            pltpu.sync_copy(rrs.at[pl.ds(0, rem)], gs_ref.at[pl.ds(offr, rem)])
            pltpu.sync_copy(rrp.at[pl.ds(0, rem)], gp_ref.at[pl.ds(offr, rem)])

    return k(hidden, sidx, pidx)


def _scatter_pallas(sp_edge, sc_edge, sidx, pidx, zeros_nd):
    e, d = sp_edge.shape
    n = zeros_nd.shape[0]
    per_t = e // NS
    nchunks = per_t // CHUNK
    rem = per_t - nchunks * CHUNK
    nzt = 10
    rpt = n // nzt

    @functools.partial(
        pl.kernel,
        out_type=[jax.ShapeDtypeStruct((n, d), jnp.float32),
                  jax.ShapeDtypeStruct((n, d), jnp.float32)],
        mesh=_sc_mesh(),
        scratch_types=[
            pltpu.VMEM_SHARED((n, d), jnp.float32),
            pltpu.VMEM((CHUNK,), jnp.int32),
            pltpu.VMEM((CHUNK,), jnp.int32),
            pltpu.VMEM((CHUNK, d), jnp.float32),
            pltpu.VMEM((CHUNK, d), jnp.float32),
            pltpu.VMEM((max(rem, 8),), jnp.int32),
            pltpu.VMEM((max(rem, 8), d), jnp.float32),
            pltpu.SemaphoreType.DMA,
            pltpu.SemaphoreType.DMA,
        ],
    )
    def k(sp_ref, sc_ref, sidx_ref, pidx_ref, z_ref, outp_ref, outc_ref,
          acc, idx0, idx1, val0, val1, ridx, rval, semi, sema):
        c = lax.axis_index("c")
        s = lax.axis_index("s")
        row0 = s * rpt

        @pl.when(s < nzt)
        def _():
            pltpu.sync_copy(z_ref.at[pl.ds(row0, rpt)], acc.at[pl.ds(row0, rpt)])

        plsc.subcore_barrier()
        base = s * per_t

        def run(val_ref, i_ref):
            idx_b = [idx0, idx1]
            val_b = [val0, val1]

            def copy_start(j, b):
                off = base + j * CHUNK
                pltpu.async_copy(i_ref.at[pl.ds(off, CHUNK)], idx_b[b], semi)
                pltpu.async_copy(val_ref.at[pl.ds(off, CHUNK)], val_b[b], semi)

            def copy_wait():
                pltpu.make_async_copy(i_ref.at[pl.ds(0, CHUNK)], idx_b[0], semi).wait()
                pltpu.make_async_copy(val_ref.at[pl.ds(0, CHUNK)], val_b[0], semi).wait()

            def add_start(b):
                pltpu.async_copy(val_b[b], acc.at[idx_b[b]], sema, add=True)

            def add_wait(b):
                pltpu.make_async_copy(val_b[b], acc.at[idx_b[b]], sema).wait()

            copy_start(0, 0)

            def body(t, carry):
                j0 = 2 * t
                copy_wait()
                add_start(0)

                @pl.when(t > 0)
                def _():
                    add_wait(1)

                copy_start(j0 + 1, 1)
                copy_wait()
                add_start(1)

                @pl.when(t < nchunks // 2 - 1)
                def _():
                    add_wait(0)
                    copy_start(j0 + 2, 0)

                return carry

            lax.fori_loop(0, nchunks // 2, body, 0)
            add_wait(0)
            add_wait(1)
            if rem:
                offr = base + nchunks * CHUNK
                pltpu.sync_copy(i_ref.at[pl.ds(offr, rem)], ridx)
                pltpu.sync_copy(val_ref.at[pl.ds(offr, rem)], rval)
                pltpu.sync_copy(rval, acc.at[ridx], add=True)

        @pl.when(c == 0)
        def _():
            run(sp_ref, sidx_ref)

        @pl.when(c == 1)
        def _():
            run(sc_ref, pidx_ref)

        plsc.subcore_barrier()

        @pl.when((s < nzt) & (c == 0))
        def _():
            pltpu.sync_copy(acc.at[pl.ds(row0, rpt)], outp_ref.at[pl.ds(row0, rpt)])

        @pl.when((s < nzt) & (c == 1))
        def _():
            pltpu.sync_copy(acc.at[pl.ds(row0, rpt)], outc_ref.at[pl.ds(row0, rpt)])

    return k(sp_edge, sc_edge, sidx, pidx, zeros_nd)


def _counts_pallas(sidx, pidx, zeros_n8, ones_c8):
    e = sidx.shape[0]
    n, d8 = zeros_n8.shape
    per_t = e // NS
    nchunks = per_t // CHUNK
    rem = per_t - nchunks * CHUNK
    nzt = 10
    rpt = n // nzt

    @functools.partial(
        pl.kernel,
        out_type=[jax.ShapeDtypeStruct((n, d8), jnp.float32),
                  jax.ShapeDtypeStruct((n, d8), jnp.float32)],
        mesh=_sc_mesh(),
        scratch_types=[
            pltpu.VMEM_SHARED((n, d8), jnp.float32),
            pltpu.VMEM((CHUNK,), jnp.int32),
            pltpu.VMEM((CHUNK,), jnp.int32),
            pltpu.VMEM((CHUNK, d8), jnp.float32),
            pltpu.VMEM((max(rem, 8),), jnp.int32),
            pltpu.SemaphoreType.DMA,
            pltpu.SemaphoreType.DMA,
        ],
    )
    def k(sidx_ref, pidx_ref, z_ref, ones_ref, outs_ref, outp_ref,
          acc, idx0, idx1, ones_v, ridx, semi, sema):
        c = lax.axis_index("c")
        s = lax.axis_index("s")
        row0 = s * rpt

        @pl.when(s < nzt)
        def _():
            pltpu.sync_copy(z_ref.at[pl.ds(row0, rpt)], acc.at[pl.ds(row0, rpt)])

        pltpu.sync_copy(ones_ref, ones_v)
        plsc.subcore_barrier()
        base = s * per_t

        def run(i_ref):
            idx_b = [idx0, idx1]

            def copy_start(j, b):
                off = base + j * CHUNK
                pltpu.async_copy(i_ref.at[pl.ds(off, CHUNK)], idx_b[b], semi)

            def copy_wait():
                pltpu.make_async_copy(i_ref.at[pl.ds(0, CHUNK)], idx_b[0], semi).wait()

            def add_start(b):
                pltpu.async_copy(ones_v, acc.at[idx_b[b]], sema, add=True)

            def add_wait(b):
                pltpu.make_async_copy(ones_v, acc.at[idx_b[b]], sema).wait()

            copy_start(0, 0)

            def body(t, carry):
                j0 = 2 * t
                copy_wait()
                add_start(0)

                @pl.when(t > 0)
                def _():
                    add_wait(1)

                copy_start(j0 + 1, 1)
                copy_wait()
                add_start(1)

                @pl.when(t < nchunks // 2 - 1)
                def _():
                    add_wait(0)
                    copy_start(j0 + 2, 0)

                return carry

            lax.fori_loop(0, nchunks // 2, body, 0)
            add_wait(0)
            add_wait(1)
            if rem:
                offr = base + nchunks * CHUNK
                pltpu.sync_copy(i_ref.at[pl.ds(offr, rem)], ridx)
                pltpu.sync_copy(ones_v.at[pl.ds(0, rem)], acc.at[ridx], add=True)

        @pl.when(c == 0)
        def _():
            run(sidx_ref)

        @pl.when(c == 1)
        def _():
            run(pidx_ref)

        plsc.subcore_barrier()

        @pl.when((s < nzt) & (c == 0))
        def _():
            pltpu.sync_copy(acc.at[pl.ds(row0, rpt)], outs_ref.at[pl.ds(row0, rpt)])

        @pl.when((s < nzt) & (c == 1))
        def _():
            pltpu.sync_copy(acc.at[pl.ds(row0, rpt)], outp_ref.at[pl.ds(row0, rpt)])

    return k(sidx, pidx, zeros_n8, ones_c8)


def kernel(batch_token, self_idx_batch, parent_idx_batch, root_mask, leaf_mask,
           start_token, end_token,
           V_W1, V_b1, V_W2, V_b2,
           E_W1, E_b1, E_W2, E_b2,
           p_W1, p_b1, p_W2, p_b2,
           c_W1, c_b1, c_W2, c_b2,
           aggr_W1, aggr_b1, aggr_W2, aggr_b2):
    n, d = batch_token.shape
    e = self_idx_batch.shape[0]
    num_hops = 2

    edge_in = jnp.maximum(jnp.maximum(jnp.ones((1, 1), jnp.float32) @ E_W1 + E_b1, 0.0)
                          @ E_W2 + E_b2, 0.0)
    edge_out = jnp.maximum(jnp.maximum(jnp.zeros((1, 1), jnp.float32) @ E_W1 + E_b1, 0.0)
                           @ E_W2 + E_b2, 0.0)
    kp = (edge_out @ p_W1[2 * d:] + p_b1).reshape(1, -1)
    kc = (edge_in @ c_W1[2 * d:] + c_b1).reshape(1, -1)

    w1cat = jnp.concatenate(
        [jnp.concatenate([p_W1[:d], p_W1[d:2 * d]], axis=0),
         jnp.concatenate([c_W1[d:2 * d], c_W1[:d]], axis=0)],
        axis=1).astype(jnp.bfloat16)

    hidden = _node_mlp(batch_token, V_W1, V_b1, V_W2, V_b2)

    zeros_nd = jnp.zeros((n, d), jnp.float32)
    ones_c = jnp.ones((CHUNK, d), jnp.float32)
    cnt_s, cnt_p = _counts_pallas(self_idx_batch, parent_idx_batch,
                                  zeros_nd, ones_c)

    rm = root_mask.reshape(n, 1)
    lm = leaf_mask.reshape(n, 1)
    st = start_token.reshape(1, d)
    et = end_token.reshape(1, d)
    pb2 = p_b2.reshape(1, -1)
    cb2 = c_b2.reshape(1, -1)

    for _ in range(num_hops):
        gs, gp = _gather_pallas(hidden, self_idx_batch, parent_idx_batch)
        sp_edge, sc_edge = _edge_mlp(gp, gs, w1cat, kp, kc, p_W2, pb2,
                                     c_W2, cb2)
        sum_p, sum_c = _scatter_pallas(sp_edge, sc_edge, self_idx_batch,
                                       parent_idx_batch, zeros_nd)
        hidden = _aggr_mlp(hidden, sum_p, sum_c, cnt_s, cnt_p, rm, lm, st, et,
                           aggr_W1, aggr_b1.reshape(1, -1), aggr_W2,
                           aggr_b2.reshape(1, -1))
    return hidden
